# baseline (speedup 1.0000x reference)
import jax
import jax.numpy as jnp
from jax.experimental import pallas as pl
from jax.experimental.pallas import tpu as pltpu

N = 10000
D = 256
DH = 128
NC = 8
ALPHA = 0.5
EPS = 1e-5


def _dot_t(a, b):
    return jax.lax.dot_general(
        a, b, (((1,), (1,)), ((), ())), preferred_element_type=jnp.float32)


def _gnn_kernel(x_ref, comm_ref,
                wv0, wo0, wp0, g0, be0,
                wv1, wo1, wp1, g1, be1,
                wv2, wo2, wp2, g2, be2,
                cemb, wvc, woc, bvc, boc,
                w1, b1, w2, b2,
                out_ref, h_ref, xv_ref, sem):
    cp = pltpu.make_async_copy(x_ref, xv_ref, sem)
    cp.start()
    def fold(wp, wo, wv):
        return jnp.dot(wp[...], jnp.dot(wo[...], wv[...],
                                        preferred_element_type=jnp.float32),
                       preferred_element_type=jnp.float32)

    a0 = fold(wp0, wo0, wv0)
    a1 = fold(wp1, wo1, wv1)
    a2 = fold(wp2, wo2, wv2)

    v8 = _dot_t(cemb[...], wvc[...]) + bvc[...]
    table = _dot_t(v8, woc[...]) + boc[...]

    def stats_to_affine(s, q, gamma, beta):
        mu = s * (1.0 / N)
        var = q * (1.0 / N) - mu * mu
        inv = jax.lax.rsqrt(var + EPS)
        scale = gamma[...] * inv
        shift = beta[...] - mu * scale
        return scale, shift

    onehot = jnp.where(
        comm_ref[...] == jax.lax.broadcasted_iota(jnp.int32, (N, NC), 1),
        jnp.float32(1.0), jnp.float32(0.0))

    att_v = jax.lax.dot_general(
        onehot, table, (((1,), (0,)), ((), ())),
        preferred_element_type=jnp.float32)

    cp.wait()
    y = _dot_t(xv_ref[...], a0)
    out_ref[...] = y
    s = jnp.sum(y, 0, keepdims=True)
    q = jnp.sum(y * y, 0, keepdims=True)
    scale0, shift0 = stats_to_affine(s, q, g0, be0)

    h1 = out_ref[...] * scale0 + shift0
    gmix = ALPHA * att_v + (1.0 - ALPHA) * h1
    h_ref[...] = gmix
    y1 = _dot_t(gmix, a1)
    out_ref[...] = y1
    s = jnp.sum(y1, 0, keepdims=True)
    q = jnp.sum(y1 * y1, 0, keepdims=True)
    scale1, shift1 = stats_to_affine(s, q, g1, be1)

    h2 = out_ref[...] * scale1 + shift1 + h_ref[...]
    gmix2 = ALPHA * att_v + (1.0 - ALPHA) * h2
    h_ref[...] = gmix2
    y2 = _dot_t(gmix2, a2)
    out_ref[...] = y2
    s = jnp.sum(y2, 0, keepdims=True)
    q = jnp.sum(y2 * y2, 0, keepdims=True)
    scale2, shift2 = stats_to_affine(s, q, g2, be2)

    h3 = out_ref[...] * scale2 + shift2 + h_ref[...]
    hid = jax.nn.relu(_dot_t(h3, w1[...]) + b1[...])
    out_ref[...] = _dot_t(hid, w2[...]) + b2[...]


def kernel(x, edge_index, communities, params):
    del edge_index
    p0, p1, p2 = params['layer0'], params['layer1'], params['layer2']
    ca = params['comm_attn']

    def row(v):
        return v.reshape(1, -1)

    comm_bcast = jnp.broadcast_to(
        communities.astype(jnp.int32).reshape(N, 1), (N, NC))

    args = (
        x, comm_bcast,
        p0['Wv'], p0['Wo'], p0['Wp'], row(p0['gamma']), row(p0['beta']),
        p1['Wv'], p1['Wo'], p1['Wp'], row(p1['gamma']), row(p1['beta']),
        p2['Wv'], p2['Wo'], p2['Wp'], row(p2['gamma']), row(p2['beta']),
        params['comm_emb'], ca['Wv'], ca['Wo'], row(ca['bv']), row(ca['bo']),
        params['pred_W1'], row(params['pred_b1']),
        params['pred_W2'], row(params['pred_b2']),
    )

    return pl.pallas_call(
        _gnn_kernel,
        out_shape=jax.ShapeDtypeStruct((N, D), jnp.float32),
        in_specs=[pl.BlockSpec(memory_space=pltpu.MemorySpace.HBM)]
        + [pl.BlockSpec(memory_space=pltpu.MemorySpace.VMEM)] * 25,
        scratch_shapes=[pltpu.VMEM((N, D), jnp.float32),
                        pltpu.VMEM((N, D), jnp.float32),
                        pltpu.SemaphoreType.DMA],
    )(*args)


# speedup vs baseline: 1.0154x; 1.0154x over previous
import jax
import jax.numpy as jnp
from jax.experimental import pallas as pl
from jax.experimental.pallas import tpu as pltpu

N = 10000
D = 256
DH = 128
NC = 8
ALPHA = 0.5
EPS = 1e-5


def _dot_t(a, b):
    return jax.lax.dot_general(
        a, b, (((1,), (1,)), ((), ())), preferred_element_type=jnp.float32)


def _gnn_kernel(x_ref, comm_ref,
                wv0, wo0, wp0, g0, be0,
                wv1, wo1, wp1, g1, be1,
                wv2, wo2, wp2, g2, be2,
                cemb, wvc, woc, bvc, boc,
                w1, b1, w2, b2,
                out_ref, h_ref, xv_ref, sem):
    cp = pltpu.make_async_copy(x_ref, xv_ref, sem)
    cp.start()
    def fold(wp, wo, wv):
        return jnp.dot(wp[...], jnp.dot(wo[...], wv[...],
                                        preferred_element_type=jnp.float32),
                       preferred_element_type=jnp.float32)

    a0 = fold(wp0, wo0, wv0)
    a1 = fold(wp1, wo1, wv1)
    a2 = fold(wp2, wo2, wv2)

    v8 = _dot_t(cemb[...], wvc[...]) + bvc[...]
    table = _dot_t(v8, woc[...]) + boc[...]

    def stats_to_affine(s, q, gamma, beta):
        mu = s * (1.0 / N)
        var = q * (1.0 / N) - mu * mu
        inv = jax.lax.rsqrt(var + EPS)
        scale = gamma[...] * inv
        shift = beta[...] - mu * scale
        return scale, shift

    onehot = jnp.where(
        comm_ref[...] == jax.lax.broadcasted_iota(jnp.int32, (N, NC), 1),
        jnp.float32(1.0), jnp.float32(0.0))

    def att():
        return jax.lax.dot_general(
            onehot, table, (((1,), (0,)), ((), ())),
            preferred_element_type=jnp.float32)

    cp.wait()
    y = _dot_t(xv_ref[...], a0)
    out_ref[...] = y
    s = jnp.sum(y, 0, keepdims=True)
    q = jnp.sum(y * y, 0, keepdims=True)
    scale0, shift0 = stats_to_affine(s, q, g0, be0)

    h1 = out_ref[...] * scale0 + shift0
    gmix = ALPHA * att() + (1.0 - ALPHA) * h1
    h_ref[...] = gmix
    y1 = _dot_t(gmix, a1)
    out_ref[...] = y1
    s = jnp.sum(y1, 0, keepdims=True)
    q = jnp.sum(y1 * y1, 0, keepdims=True)
    scale1, shift1 = stats_to_affine(s, q, g1, be1)

    h2 = out_ref[...] * scale1 + shift1 + h_ref[...]
    gmix2 = ALPHA * att() + (1.0 - ALPHA) * h2
    h_ref[...] = gmix2
    y2 = _dot_t(gmix2, a2)
    out_ref[...] = y2
    s = jnp.sum(y2, 0, keepdims=True)
    q = jnp.sum(y2 * y2, 0, keepdims=True)
    scale2, shift2 = stats_to_affine(s, q, g2, be2)

    h3 = out_ref[...] * scale2 + shift2 + h_ref[...]
    hid = jax.nn.relu(_dot_t(h3, w1[...]) + b1[...])
    out_ref[...] = _dot_t(hid, w2[...]) + b2[...]


def kernel(x, edge_index, communities, params):
    del edge_index
    p0, p1, p2 = params['layer0'], params['layer1'], params['layer2']
    ca = params['comm_attn']

    def row(v):
        return v.reshape(1, -1)

    comm_bcast = jnp.broadcast_to(
        communities.astype(jnp.int32).reshape(N, 1), (N, NC))

    args = (
        x, comm_bcast,
        p0['Wv'], p0['Wo'], p0['Wp'], row(p0['gamma']), row(p0['beta']),
        p1['Wv'], p1['Wo'], p1['Wp'], row(p1['gamma']), row(p1['beta']),
        p2['Wv'], p2['Wo'], p2['Wp'], row(p2['gamma']), row(p2['beta']),
        params['comm_emb'], ca['Wv'], ca['Wo'], row(ca['bv']), row(ca['bo']),
        params['pred_W1'], row(params['pred_b1']),
        params['pred_W2'], row(params['pred_b2']),
    )

    return pl.pallas_call(
        _gnn_kernel,
        out_shape=jax.ShapeDtypeStruct((N, D), jnp.float32),
        in_specs=[pl.BlockSpec(memory_space=pltpu.MemorySpace.HBM)]
        + [pl.BlockSpec(memory_space=pltpu.MemorySpace.VMEM)] * 25,
        scratch_shapes=[pltpu.VMEM((N, D), jnp.float32),
                        pltpu.VMEM((N, D), jnp.float32),
                        pltpu.SemaphoreType.DMA],
    )(*args)


# speedup vs baseline: 1.0182x; 1.0027x over previous
import jax
import jax.numpy as jnp
from jax.experimental import pallas as pl
from jax.experimental.pallas import tpu as pltpu

N = 10000
D = 256
DH = 128
NC = 8
ALPHA = 0.5
EPS = 1e-5


def _dot_t(a, b):
    return jax.lax.dot_general(
        a, b, (((1,), (1,)), ((), ())), preferred_element_type=jnp.float32)


def _gnn_kernel(x_ref, comm_ref,
                wv0, wo0, wp0, g0, be0,
                wv1, wo1, wp1, g1, be1,
                wv2, wo2, wp2, g2, be2,
                cemb, wvc, woc, bvc, boc,
                w1, b1, w2, b2,
                out_ref, h_ref):
    def fold(wp, wo, wv):
        return jnp.dot(wp[...], jnp.dot(wo[...], wv[...],
                                        preferred_element_type=jnp.float32),
                       preferred_element_type=jnp.float32)

    a0 = fold(wp0, wo0, wv0)
    a1 = fold(wp1, wo1, wv1)
    a2 = fold(wp2, wo2, wv2)

    v8 = _dot_t(cemb[...], wvc[...]) + bvc[...]
    table = _dot_t(v8, woc[...]) + boc[...]

    def stats_to_affine(s, q, gamma, beta):
        mu = s * (1.0 / N)
        var = q * (1.0 / N) - mu * mu
        inv = jax.lax.rsqrt(var + EPS)
        scale = gamma[...] * inv
        shift = beta[...] - mu * scale
        return scale, shift

    H = N // 2
    halves = (pl.ds(0, H), pl.ds(H, H))

    def att_half(d):
        cc = comm_ref[d, :]
        oh = jnp.where(
            cc == jax.lax.broadcasted_iota(jnp.int32, (H, NC), 1),
            jnp.float32(1.0), jnp.float32(0.0))
        return jax.lax.dot_general(
            oh, table, (((1,), (0,)), ((), ())),
            preferred_element_type=jnp.float32)

    s = q = 0.0
    for d in halves:
        y = _dot_t(x_ref[d, :], a0)
        out_ref[d, :] = y
        s = s + jnp.sum(y, 0, keepdims=True)
        q = q + jnp.sum(y * y, 0, keepdims=True)
    scale0, shift0 = stats_to_affine(s, q, g0, be0)

    s = q = 0.0
    for d in halves:
        h1 = out_ref[d, :] * scale0 + shift0
        gmix = ALPHA * att_half(d) + (1.0 - ALPHA) * h1
        h_ref[d, :] = gmix
        y1 = _dot_t(gmix, a1)
        out_ref[d, :] = y1
        s = s + jnp.sum(y1, 0, keepdims=True)
        q = q + jnp.sum(y1 * y1, 0, keepdims=True)
    scale1, shift1 = stats_to_affine(s, q, g1, be1)

    s = q = 0.0
    for d in halves:
        h2 = out_ref[d, :] * scale1 + shift1 + h_ref[d, :]
        gmix2 = ALPHA * att_half(d) + (1.0 - ALPHA) * h2
        h_ref[d, :] = gmix2
        y2 = _dot_t(gmix2, a2)
        out_ref[d, :] = y2
        s = s + jnp.sum(y2, 0, keepdims=True)
        q = q + jnp.sum(y2 * y2, 0, keepdims=True)
    scale2, shift2 = stats_to_affine(s, q, g2, be2)

    for d in halves:
        h3 = out_ref[d, :] * scale2 + shift2 + h_ref[d, :]
        hid = jax.nn.relu(_dot_t(h3, w1[...]) + b1[...])
        out_ref[d, :] = _dot_t(hid, w2[...]) + b2[...]


def kernel(x, edge_index, communities, params):
    del edge_index
    p0, p1, p2 = params['layer0'], params['layer1'], params['layer2']
    ca = params['comm_attn']

    def row(v):
        return v.reshape(1, -1)

    comm_bcast = jnp.broadcast_to(
        communities.astype(jnp.int32).reshape(N, 1), (N, NC))

    args = (
        x, comm_bcast,
        p0['Wv'], p0['Wo'], p0['Wp'], row(p0['gamma']), row(p0['beta']),
        p1['Wv'], p1['Wo'], p1['Wp'], row(p1['gamma']), row(p1['beta']),
        p2['Wv'], p2['Wo'], p2['Wp'], row(p2['gamma']), row(p2['beta']),
        params['comm_emb'], ca['Wv'], ca['Wo'], row(ca['bv']), row(ca['bo']),
        params['pred_W1'], row(params['pred_b1']),
        params['pred_W2'], row(params['pred_b2']),
    )

    return pl.pallas_call(
        _gnn_kernel,
        out_shape=jax.ShapeDtypeStruct((N, D), jnp.float32),
        scratch_shapes=[pltpu.VMEM((N, D), jnp.float32)],
    )(*args)


# speedup vs baseline: 1.0338x; 1.0153x over previous
import jax
import jax.numpy as jnp
from jax.experimental import pallas as pl
from jax.experimental.pallas import tpu as pltpu

N = 10000
D = 256
DH = 128
NC = 8
ALPHA = 0.5
EPS = 1e-5


def _dot_t(a, b):
    return jax.lax.dot_general(
        a, b, (((1,), (1,)), ((), ())), preferred_element_type=jnp.float32)


def _gnn_kernel(x_ref, comm_ref,
                wv0, wo0, wp0, g0, be0,
                wv1, wo1, wp1, g1, be1,
                wv2, wo2, wp2, g2, be2,
                cemb, wvc, woc, bvc, boc,
                w1, b1, w2, b2,
                out_ref, h_ref):
    def fold(wp, wo, wv):
        return jnp.dot(wp[...], jnp.dot(wo[...], wv[...],
                                        preferred_element_type=jnp.float32),
                       preferred_element_type=jnp.float32)

    a0 = fold(wp0, wo0, wv0)
    a1 = fold(wp1, wo1, wv1)
    a2 = fold(wp2, wo2, wv2)

    v8 = _dot_t(cemb[...], wvc[...]) + bvc[...]
    table = _dot_t(v8, woc[...]) + boc[...]

    def stats_to_affine(s, q, gamma, beta):
        mu = s * (1.0 / N)
        var = q * (1.0 / N) - mu * mu
        inv = jax.lax.rsqrt(var + EPS)
        scale = gamma[...] * inv
        shift = beta[...] - mu * scale
        return scale, shift

    onehot = jnp.where(
        comm_ref[...] == jax.lax.broadcasted_iota(jnp.int32, (N, NC), 1),
        jnp.float32(1.0), jnp.float32(0.0))

    def att():
        return jax.lax.dot_general(
            onehot, table, (((1,), (0,)), ((), ())),
            preferred_element_type=jnp.float32)

    y = _dot_t(x_ref[...], a0)
    out_ref[...] = y
    s = jnp.sum(y, 0, keepdims=True)
    q = jnp.sum(y * y, 0, keepdims=True)
    scale0, shift0 = stats_to_affine(s, q, g0, be0)

    h1 = out_ref[...] * scale0 + shift0
    gmix = ALPHA * att() + (1.0 - ALPHA) * h1
    h_ref[...] = gmix
    y1 = _dot_t(gmix, a1)
    out_ref[...] = y1
    s = jnp.sum(y1, 0, keepdims=True)
    q = jnp.sum(y1 * y1, 0, keepdims=True)
    scale1, shift1 = stats_to_affine(s, q, g1, be1)

    h2 = out_ref[...] * scale1 + shift1 + h_ref[...]
    gmix2 = ALPHA * att() + (1.0 - ALPHA) * h2
    h_ref[...] = gmix2
    y2 = _dot_t(gmix2, a2)
    out_ref[...] = y2
    s = jnp.sum(y2, 0, keepdims=True)
    q = jnp.sum(y2 * y2, 0, keepdims=True)
    scale2, shift2 = stats_to_affine(s, q, g2, be2)

    h3 = out_ref[...] * scale2 + shift2 + h_ref[...]
    hid = jax.nn.relu(_dot_t(h3, w1[...]) + b1[...])
    out_ref[...] = _dot_t(hid, w2[...]) + b2[...]


def kernel(x, edge_index, communities, params):
    del edge_index
    p0, p1, p2 = params['layer0'], params['layer1'], params['layer2']
    ca = params['comm_attn']

    def row(v):
        return v.reshape(1, -1)

    comm_bcast = jnp.broadcast_to(
        communities.astype(jnp.int32).reshape(N, 1), (N, NC))

    args = (
        x, comm_bcast,
        p0['Wv'], p0['Wo'], p0['Wp'], row(p0['gamma']), row(p0['beta']),
        p1['Wv'], p1['Wo'], p1['Wp'], row(p1['gamma']), row(p1['beta']),
        p2['Wv'], p2['Wo'], p2['Wp'], row(p2['gamma']), row(p2['beta']),
        params['comm_emb'], ca['Wv'], ca['Wo'], row(ca['bv']), row(ca['bo']),
        params['pred_W1'], row(params['pred_b1']),
        params['pred_W2'], row(params['pred_b2']),
    )

    return pl.pallas_call(
        _gnn_kernel,
        out_shape=jax.ShapeDtypeStruct((N, D), jnp.float32),
        scratch_shapes=[pltpu.VMEM((N, D), jnp.float32)],
    )(*args)
